# Initial kernel scaffold; baseline (speedup 1.0000x reference)
import jax
import jax.numpy as jnp
from jax.experimental import pallas as pl


def kernel(user_idx, item_idx, P, Q, b_u, b_i):
    raise NotImplementedError("write your pallas kernel here")


# speedup vs baseline: 1.0724x; 1.0724x over previous
import functools

import jax
import jax.numpy as jnp
from jax import lax
from jax.experimental import pallas as pl
from jax.experimental.pallas import tpu as pltpu
from jax.experimental.pallas import tpu_sc as plsc

_NC = 2
_NS = 16
_L = 16
_MEAN = 3.5


@functools.lru_cache(maxsize=None)
def _build(B, K):
    NW = _NC * _NS
    BPW = B // NW
    CH = min(BPW, 128)
    NCH = BPW // CH
    mesh = plsc.VectorSubcoreMesh(
        core_axis_name="c", subcore_axis_name="s",
        num_cores=_NC, num_subcores=_NS)

    def body(u_hbm, i_hbm, p_hbm, q_hbm, bu_hbm, bi_hbm, out_hbm,
             uidx_v, iidx_v, p_v, q_v, bu_v, bi_v, out_v,
             sem_rows, sem_bias):
        wid = lax.axis_index("s") * _NC + lax.axis_index("c")
        base = wid * BPW
        lanes = lax.iota(jnp.int32, _L)

        for c in range(NCH):
            pltpu.sync_copy(u_hbm.at[pl.ds(base + c * CH, CH)], uidx_v.at[c])
            pltpu.sync_copy(i_hbm.at[pl.ds(base + c * CH, CH)], iidx_v.at[c])

        bias_dmas = []
        for c in range(NCH):
            dbu = pltpu.make_async_copy(bu_hbm.at[uidx_v.at[c]], bu_v.at[c],
                                        sem_bias)
            dbi = pltpu.make_async_copy(bi_hbm.at[iidx_v.at[c]], bi_v.at[c],
                                        sem_bias)
            dbu.start()
            dbi.start()
            bias_dmas.append((dbu, dbi))
        for dbu, dbi in bias_dmas:
            dbu.wait()
            dbi.wait()

        for c in range(NCH):
            dp = pltpu.make_async_copy(p_hbm.at[uidx_v.at[c]], p_v, sem_rows)
            dq = pltpu.make_async_copy(q_hbm.at[iidx_v.at[c]], q_v, sem_rows)
            dp.start()
            dq.start()
            dp.wait()
            dq.wait()

            def group(g, carry, c=c):
                red = jnp.zeros((_L,), jnp.float32)
                for l in range(_L):
                    e = g * _L + l
                    acc = p_v[e, pl.ds(0, _L)] * q_v[e, pl.ds(0, _L)]
                    for j in range(1, K // _L):
                        acc = acc + (p_v[e, pl.ds(j * _L, _L)] *
                                     q_v[e, pl.ds(j * _L, _L)])
                    red = jnp.where(lanes == l, jnp.sum(acc), red)
                red = red + bu_v[c, pl.ds(g * _L, _L)]
                red = red + bi_v[c, pl.ds(g * _L, _L)]
                out_v[pl.ds(c * CH + g * _L, _L)] = red + _MEAN
                return carry

            lax.fori_loop(0, CH // _L, group, 0)

        pltpu.sync_copy(out_v, out_hbm.at[pl.ds(base, BPW)])

    return pl.kernel(
        body,
        out_type=jax.ShapeDtypeStruct((B,), jnp.float32),
        mesh=mesh,
        compiler_params=pltpu.CompilerParams(needs_layout_passes=False),
        scratch_types=[
            pltpu.VMEM((NCH, CH), jnp.int32),
            pltpu.VMEM((NCH, CH), jnp.int32),
            pltpu.VMEM((CH, K), jnp.float32),
            pltpu.VMEM((CH, K), jnp.float32),
            pltpu.VMEM((NCH, CH), jnp.float32),
            pltpu.VMEM((NCH, CH), jnp.float32),
            pltpu.VMEM((BPW,), jnp.float32),
            pltpu.SemaphoreType.DMA,
            pltpu.SemaphoreType.DMA,
        ],
    )


def kernel(user_idx, item_idx, P, Q, b_u, b_i):
    B = user_idx.shape[0]
    K = P.shape[1]
    fn = _build(B, K)
    return fn(user_idx.astype(jnp.int32), item_idx.astype(jnp.int32),
              P, Q, b_u.reshape(-1), b_i.reshape(-1))


# speedup vs baseline: 1.7246x; 1.6081x over previous
import functools

import jax
import jax.numpy as jnp
from jax import lax
from jax.experimental import pallas as pl
from jax.experimental.pallas import tpu as pltpu
from jax.experimental.pallas import tpu_sc as plsc

_NC = 2
_NS = 16
_L = 16
_PAD = _L + 1
_MEAN = 3.5


@functools.lru_cache(maxsize=None)
def _build(B, K):
    NW = _NC * _NS
    BPW = B // NW
    CH = min(BPW, 128)
    NCH = BPW // CH
    NBUF = min(NCH, 2)
    mesh = plsc.VectorSubcoreMesh(
        core_axis_name="c", subcore_axis_name="s",
        num_cores=_NC, num_subcores=_NS)

    def body(u_hbm, i_hbm, p_hbm, q_hbm, bu_hbm, bi_hbm, out_hbm,
             uidx_v, iidx_v, p_v, q_v, bu_v, bi_v, out_v, tmp_v,
             sem_idx, sem_bias, *sem_rows):
        wid = lax.axis_index("s") * _NC + lax.axis_index("c")
        base = wid * BPW
        lanes = lax.iota(jnp.int32, _L)
        rowbase = lanes * _PAD

        idx_dmas = []
        for c in range(NCH):
            du = pltpu.make_async_copy(
                u_hbm.at[pl.ds(base + c * CH, CH)], uidx_v.at[c], sem_idx)
            di = pltpu.make_async_copy(
                i_hbm.at[pl.ds(base + c * CH, CH)], iidx_v.at[c], sem_idx)
            du.start()
            di.start()
            idx_dmas.append((du, di))
        for du, di in idx_dmas:
            du.wait()
            di.wait()

        bias_dmas = []
        for c in range(NCH):
            dbu = pltpu.make_async_copy(bu_hbm.at[uidx_v.at[c]], bu_v.at[c],
                                        sem_bias)
            dbi = pltpu.make_async_copy(bi_hbm.at[iidx_v.at[c]], bi_v.at[c],
                                        sem_bias)
            dbu.start()
            dbi.start()
            bias_dmas.append((dbu, dbi))

        def fire(c):
            b = c % NBUF
            dp = pltpu.make_async_copy(p_hbm.at[uidx_v.at[c]], p_v.at[b],
                                       sem_rows[2 * b])
            dq = pltpu.make_async_copy(q_hbm.at[iidx_v.at[c]], q_v.at[b],
                                       sem_rows[2 * b + 1])
            dp.start()
            dq.start()
            return dp, dq

        row_dmas = {0: fire(0)}
        for c in range(NCH):
            if c + 1 < NCH:
                row_dmas[c + 1] = fire(c + 1)
            dp, dq = row_dmas.pop(c)
            dp.wait()
            dq.wait()
            if c == 0:
                for dbu, dbi in bias_dmas:
                    dbu.wait()
                    dbi.wait()
            b = c % NBUF
            pb = p_v.at[b]
            qb = q_v.at[b]

            def group(g, carry, c=c, pb=pb, qb=qb):
                for l in range(_L):
                    e = g * _L + l
                    acc = pb[e, pl.ds(0, _L)] * qb[e, pl.ds(0, _L)]
                    for j in range(1, K // _L):
                        acc = acc + (pb[e, pl.ds(j * _L, _L)] *
                                     qb[e, pl.ds(j * _L, _L)])
                    tmp_v[pl.ds(l * _PAD, _L)] = acc
                red = plsc.load_gather(tmp_v, [rowbase])
                for j in range(1, _L):
                    red = red + plsc.load_gather(tmp_v, [rowbase + j])
                red = red + bu_v[c, pl.ds(g * _L, _L)]
                red = red + bi_v[c, pl.ds(g * _L, _L)]
                out_v[pl.ds(c * CH + g * _L, _L)] = red + _MEAN
                return carry

            lax.fori_loop(0, CH // _L, group, 0)

        pltpu.sync_copy(out_v, out_hbm.at[pl.ds(base, BPW)])

    return pl.kernel(
        body,
        out_type=jax.ShapeDtypeStruct((B,), jnp.float32),
        mesh=mesh,
        compiler_params=pltpu.CompilerParams(needs_layout_passes=False),
        scratch_types=[
            pltpu.VMEM((NCH, CH), jnp.int32),
            pltpu.VMEM((NCH, CH), jnp.int32),
            pltpu.VMEM((NBUF, CH, K), jnp.float32),
            pltpu.VMEM((NBUF, CH, K), jnp.float32),
            pltpu.VMEM((NCH, CH), jnp.float32),
            pltpu.VMEM((NCH, CH), jnp.float32),
            pltpu.VMEM((BPW,), jnp.float32),
            pltpu.VMEM((_L * _PAD,), jnp.float32),
            pltpu.SemaphoreType.DMA,
            pltpu.SemaphoreType.DMA,
            pltpu.SemaphoreType.DMA,
            pltpu.SemaphoreType.DMA,
            pltpu.SemaphoreType.DMA,
            pltpu.SemaphoreType.DMA,
        ],
    )


def kernel(user_idx, item_idx, P, Q, b_u, b_i):
    B = user_idx.shape[0]
    K = P.shape[1]
    fn = _build(B, K)
    return fn(user_idx.astype(jnp.int32), item_idx.astype(jnp.int32),
              P, Q, b_u.reshape(-1), b_i.reshape(-1))
